# baseline (speedup 1.0000x reference)
import jax
import jax.numpy as jnp
from jax.experimental import pallas as pl

_BOUNDS = ((0, 2736), (2736, 5472), (5472, 8192))


def _vq_argmin_kernel(x_ref, e_ref, out_ref):
    x = x_ref[...]
    mblk = x.shape[0]
    x_sq = jnp.sum(x * x, axis=1, keepdims=True)
    v = jnp.full((mblk,), jnp.inf, dtype=jnp.float32)
    idx = jnp.zeros((mblk,), dtype=jnp.int32)
    for lo, hi in _BOUNDS:
        w = hi - lo
        e_c = e_ref[lo:hi, :]
        e_sq_c = jnp.sum(e_c * e_c, axis=1)
        dt = jax.lax.dot_general(
            x, e_c, (((1,), (1,)), ((), ())),
            preferred_element_type=jnp.float32,
        )
        dc = (x_sq - 2.0 * dt) + e_sq_c[None, :]
        lane = jax.lax.broadcasted_iota(jnp.int32, (mblk, w), 1)
        m_c = jnp.min(dc, axis=1)
        i_c = jnp.min(
            jnp.where(dc == m_c[:, None], lane, jnp.int32(0x7FFFFFFF)),
            axis=1,
        ) + lo
        steal = m_c < v
        idx = jnp.where(steal, i_c, idx)
        v = jnp.where(steal, m_c, v).astype(jnp.bfloat16).astype(jnp.float32)
    out_ref[0, 0, :] = idx


def kernel(z_e_x, embedding_weight):
    B, D, T = z_e_x.shape
    Kk = embedding_weight.shape[0]
    M = B * T
    flat = jnp.transpose(z_e_x, (0, 2, 1)).reshape(M, D)
    Mblk = 256
    nblk = M // Mblk
    out = pl.pallas_call(
        _vq_argmin_kernel,
        grid=(nblk,),
        in_specs=[
            pl.BlockSpec((Mblk, D), lambda i: (i, 0)),
            pl.BlockSpec((Kk, D), lambda i: (0, 0)),
        ],
        out_specs=pl.BlockSpec((1, 1, Mblk), lambda i: (i, 0, 0)),
        out_shape=jax.ShapeDtypeStruct((nblk, 1, Mblk), jnp.int32),
    )(flat, embedding_weight)
    return out.reshape(B, T)


# speedup vs baseline: 1.2243x; 1.2243x over previous
import jax
import jax.numpy as jnp
from jax.experimental import pallas as pl
from jax.experimental.pallas import tpu as pltpu

_BOUNDS = ((0, 2736), (2736, 5472), (5472, 8192))


def _vq_argmin_kernel(x_ref, e_ref, out_ref):
    x = x_ref[...]
    mblk = x.shape[0]
    x_sq = jnp.sum(x * x, axis=1, keepdims=True)
    v = jnp.full((mblk,), jnp.inf, dtype=jnp.float32)
    idx = jnp.zeros((mblk,), dtype=jnp.int32)
    for lo, hi in _BOUNDS:
        w = hi - lo
        e_c = e_ref[lo:hi, :]
        e_sq_c = jnp.sum(e_c * e_c, axis=1)
        dt = jax.lax.dot_general(
            x, e_c, (((1,), (1,)), ((), ())),
            preferred_element_type=jnp.float32,
        )
        dc = (x_sq - 2.0 * dt) + e_sq_c[None, :]
        lane = jax.lax.broadcasted_iota(jnp.int32, (mblk, w), 1)
        m_c = jnp.min(dc, axis=1)
        i_c = jnp.min(
            jnp.where(dc == m_c[:, None], lane, jnp.int32(0x7FFFFFFF)),
            axis=1,
        ) + lo
        steal = m_c < v
        idx = jnp.where(steal, i_c, idx)
        v = jnp.where(steal, m_c, v).astype(jnp.bfloat16).astype(jnp.float32)
    out_ref[0, 0, :] = idx


def kernel(z_e_x, embedding_weight):
    B, D, T = z_e_x.shape
    Kk = embedding_weight.shape[0]
    M = B * T
    flat = jnp.transpose(z_e_x, (0, 2, 1)).reshape(M, D)
    Mblk = 512
    nblk = M // Mblk
    out = pl.pallas_call(
        _vq_argmin_kernel,
        grid=(nblk,),
        in_specs=[
            pl.BlockSpec((Mblk, D), lambda i: (i, 0)),
            pl.BlockSpec((Kk, D), lambda i: (0, 0)),
        ],
        out_specs=pl.BlockSpec((1, 1, Mblk), lambda i: (i, 0, 0)),
        out_shape=jax.ShapeDtypeStruct((nblk, 1, Mblk), jnp.int32),
        compiler_params=pltpu.CompilerParams(
            dimension_semantics=("parallel",),
        ),
    )(flat, embedding_weight)
    return out.reshape(B, T)


# speedup vs baseline: 1.3548x; 1.1066x over previous
import jax
import jax.numpy as jnp
from jax.experimental import pallas as pl
from jax.experimental.pallas import tpu as pltpu

_BOUNDS = ((0, 2736), (2736, 5472), (5472, 8192))


def _vq_argmin_kernel(x_ref, e_ref, out_ref):
    x = x_ref[...]
    mblk = x.shape[0]
    x_sq = jnp.sum(x * x, axis=1, keepdims=True)
    v = jnp.full((mblk,), jnp.inf, dtype=jnp.float32)
    idx = jnp.zeros((mblk,), dtype=jnp.int32)
    for lo, hi in _BOUNDS:
        w = hi - lo
        e_c = e_ref[lo:hi, :]
        e_sq_c = jnp.sum(e_c * e_c, axis=1)
        dt = jax.lax.dot_general(
            x, e_c, (((1,), (1,)), ((), ())),
            preferred_element_type=jnp.float32,
        )
        dc = (x_sq - 2.0 * dt) + e_sq_c[None, :]
        lane = jax.lax.broadcasted_iota(jnp.int32, (mblk, w), 1)
        m_c = jnp.min(dc, axis=1)
        i_c = jnp.min(
            jnp.where(dc == m_c[:, None], lane, jnp.int32(0x7FFFFFFF)),
            axis=1,
        ) + lo
        steal = m_c < v
        idx = jnp.where(steal, i_c, idx)
        v = jnp.where(steal, m_c, v).astype(jnp.bfloat16).astype(jnp.float32)
    out_ref[0, 0, :] = idx


def kernel(z_e_x, embedding_weight):
    B, D, T = z_e_x.shape
    Kk = embedding_weight.shape[0]
    M = B * T
    flat = jnp.transpose(z_e_x, (0, 2, 1)).reshape(M, D)
    Mblk = 1024
    nblk = M // Mblk
    out = pl.pallas_call(
        _vq_argmin_kernel,
        grid=(nblk,),
        in_specs=[
            pl.BlockSpec((Mblk, D), lambda i: (i, 0)),
            pl.BlockSpec((Kk, D), lambda i: (0, 0)),
        ],
        out_specs=pl.BlockSpec((1, 1, Mblk), lambda i: (i, 0, 0)),
        out_shape=jax.ShapeDtypeStruct((nblk, 1, Mblk), jnp.int32),
        compiler_params=pltpu.CompilerParams(
            dimension_semantics=("parallel",),
        ),
    )(flat, embedding_weight)
    return out.reshape(B, T)


# speedup vs baseline: 1.4804x; 1.0927x over previous
import jax
import jax.numpy as jnp
from jax.experimental import pallas as pl
from jax.experimental.pallas import tpu as pltpu

_BOUNDS = ((0, 2736), (2736, 5472), (5472, 8192))


def _vq_argmin_kernel(x_ref, e_ref, out_ref):
    x = x_ref[...]
    mblk = x.shape[0]
    x_sq = jnp.sum(x * x, axis=1, keepdims=True)
    v = jnp.full((mblk,), jnp.inf, dtype=jnp.float32)
    idx = jnp.zeros((mblk,), dtype=jnp.int32)
    for lo, hi in _BOUNDS:
        w = hi - lo
        e_c = e_ref[lo:hi, :]
        e_sq_c = jnp.sum(e_c * e_c, axis=1)
        dt = jax.lax.dot_general(
            x, e_c, (((1,), (1,)), ((), ())),
            preferred_element_type=jnp.float32,
        )
        dc = (x_sq - 2.0 * dt) + e_sq_c[None, :]
        lane = jax.lax.broadcasted_iota(jnp.int32, (mblk, w), 1)
        m_c = jnp.min(dc, axis=1)
        i_c = jnp.min(
            jnp.where(dc == m_c[:, None], lane, jnp.int32(0x7FFFFFFF)),
            axis=1,
        ) + lo
        steal = m_c < v
        idx = jnp.where(steal, i_c, idx)
        v = jnp.where(steal, m_c, v).astype(jnp.bfloat16).astype(jnp.float32)
    out_ref[0, 0, :] = idx


def kernel(z_e_x, embedding_weight):
    B, D, T = z_e_x.shape
    Kk = embedding_weight.shape[0]
    M = B * T
    flat = jnp.transpose(z_e_x, (0, 2, 1)).reshape(M, D)
    Mblk = 2048
    nblk = M // Mblk
    out = pl.pallas_call(
        _vq_argmin_kernel,
        grid=(nblk,),
        in_specs=[
            pl.BlockSpec((Mblk, D), lambda i: (i, 0)),
            pl.BlockSpec((Kk, D), lambda i: (0, 0)),
        ],
        out_specs=pl.BlockSpec((1, 1, Mblk), lambda i: (i, 0, 0)),
        out_shape=jax.ShapeDtypeStruct((nblk, 1, Mblk), jnp.int32),
        compiler_params=pltpu.CompilerParams(
            dimension_semantics=("parallel",),
        ),
    )(flat, embedding_weight)
    return out.reshape(B, T)
